# Initial kernel scaffold; baseline (speedup 1.0000x reference)
import jax
import jax.numpy as jnp
from jax.experimental import pallas as pl


def kernel(x_i, x_j, edge_index_ii, edge_index_jj, edge_index_ij, edge_index_ji, edge_attr_ii, edge_attr_jj, edge_attr_ij, edge_attr_ji, batch_i, batch_j, W1, b1, W2, b2, root_ii, bias_ii, root_jj, bias_jj, root_ij, bias_ij, root_ji, bias_ji, Wli, bli, Wlj, blj, Wlin, blin):
    raise NotImplementedError("write your pallas kernel here")


# speedup vs baseline: 2.4519x; 2.4519x over previous
import functools

import jax
import jax.numpy as jnp
from jax import lax
from jax.experimental import pallas as pl
from jax.experimental.pallas import tpu as pltpu
from jax.experimental.pallas import tpu_sc as plsc

N = 10000
E = 160000
DIM = 16
B = 128

NW = 32
EPW = 20480
EP = NW * EPW
NCHUNK = EPW // 128
NP = 10240
NB = 4 * NP

def _mesh():
    return plsc.VectorSubcoreMesh(core_axis_name="c", subcore_axis_name="s")


_sc_params = pltpu.CompilerParams(use_tc_tiling_on_sc=False)
_sc_params_nl = pltpu.CompilerParams(
    use_tc_tiling_on_sc=False, needs_layout_passes=False
)


def _sc_gather(table, idx3):
    @functools.partial(
        pl.kernel,
        mesh=_mesh(),
        out_type=jax.ShapeDtypeStruct((EP, 16), jnp.float32),
        compiler_params=_sc_params,
        scratch_types=[
            pltpu.VMEM((NCHUNK, 128), jnp.int32),
            pltpu.VMEM((2048, 16), jnp.float32),
            pltpu.SemaphoreType.DMA,
        ],
    )
    def gk(table_hbm, idx_hbm, out_hbm, idx_v, rows_v, sem):
        wid = lax.axis_index("s") * 2 + lax.axis_index("c")
        pltpu.sync_copy(idx_hbm.at[wid], idx_v)

        @pl.loop(0, NCHUNK // 16)
        def _(ob):
            copies = []
            for j in range(16):
                copies.append(
                    pltpu.async_copy(
                        table_hbm.at[idx_v.at[ob * 16 + j]],
                        rows_v.at[pl.ds(j * 128, 128)],
                        sem,
                    )
                )
            for cp in copies:
                cp.wait()
            pltpu.sync_copy(
                rows_v, out_hbm.at[pl.ds(wid * EPW + ob * 2048, 2048)]
            )

    return gk(table, idx3)


def _sc_scatter(ms, dst):
    CH = 2560

    @functools.partial(
        pl.kernel,
        mesh=_mesh(),
        out_type=jax.ShapeDtypeStruct((NW, 2, NB), jnp.float32),
        compiler_params=_sc_params_nl,
        scratch_types=[
            pltpu.VMEM((CH,), jnp.int32),
            pltpu.VMEM((CH,), jnp.float32),
            pltpu.VMEM((NB,), jnp.float32),
            pltpu.VMEM((NB,), jnp.float32),
        ],
    )
    def sk(ms_hbm, dst_hbm, out_hbm, idx_v, vals_v, s_acc, c_acc):
        c = lax.axis_index("c")
        s = lax.axis_index("s")
        wid = s * 2 + c
        zero16 = jnp.zeros((16,), jnp.float32)
        ones16 = jnp.ones((16,), jnp.float32)

        @pl.loop(0, NB, step=16)
        def _(i):
            s_acc[pl.ds(i, 16)] = zero16
            c_acc[pl.ds(i, 16)] = zero16

        @pl.loop(0, EPW // CH)
        def _(ob):
            base = wid * EPW + ob * CH
            pltpu.sync_copy(dst_hbm.at[pl.ds(base, CH)], idx_v)
            pltpu.sync_copy(ms_hbm.at[pl.ds(base, CH)], vals_v)

            @pl.loop(0, CH, step=16)
            def _(e):
                iv = idx_v[pl.ds(e, 16)]
                plsc.addupdate_scatter(s_acc, [iv], vals_v[pl.ds(e, 16)])
                plsc.addupdate_scatter(c_acc, [iv], ones16)

        pltpu.sync_copy(s_acc, out_hbm.at[wid, 0])
        pltpu.sync_copy(c_acc, out_hbm.at[wid, 1])

    return sk(ms, dst)


def _tc_mlp(ea, xs, w1, b1, w2, b2):
    BE = 10240
    G = EP // BE

    def body(ea_ref, xs_ref, w1_ref, b1_ref, w2_ref, b2_ref, out_ref):
        sel = (
            lax.broadcasted_iota(jnp.int32, (256, 16), 0) // 16
            == lax.broadcasted_iota(jnp.int32, (256, 16), 1)
        ).astype(jnp.float32)
        w2r = jnp.dot(w2_ref[...], sel, preferred_element_type=jnp.float32, precision=lax.Precision.HIGHEST)
        b2r = jnp.dot(b2_ref[...], sel, preferred_element_type=jnp.float32, precision=lax.Precision.HIGHEST)
        g = jnp.maximum(
            jnp.dot(ea_ref[...], w1_ref[...], preferred_element_type=jnp.float32, precision=lax.Precision.HIGHEST)
            + b1_ref[...],
            0.0,
        )
        q = jnp.dot(g, w2r, preferred_element_type=jnp.float32, precision=lax.Precision.HIGHEST) + b2r
        out_ref[...] = jnp.sum(q * xs_ref[...], axis=1, keepdims=True)

    return pl.pallas_call(
        body,
        grid=(G,),
        in_specs=[
            pl.BlockSpec((BE, 8), lambda i: (i, 0)),
            pl.BlockSpec((BE, 16), lambda i: (i, 0)),
            pl.BlockSpec((8, 128), lambda i: (0, 0)),
            pl.BlockSpec((1, 128), lambda i: (0, 0)),
            pl.BlockSpec((128, 256), lambda i: (0, 0)),
            pl.BlockSpec((1, 256), lambda i: (0, 0)),
        ],
        out_specs=pl.BlockSpec((BE, 1), lambda i: (i, 0)),
        out_shape=jax.ShapeDtypeStruct((EP, 1), jnp.float32),
    )(ea, xs, w1, b1, w2, b2)


def _tc_final(xiT, xjT, sc4, bci, bcj, riiT, rjiT, rjjT, rijT, bii, bji, bjj,
              bij, wliT, bliT, wljT, bljT, wlin, blin):
    def body(xiT_ref, xjT_ref, sc_ref, bci_ref, bcj_ref, rii_ref, rji_ref,
             rjj_ref, rij_ref, bii_ref, bji_ref, bjj_ref, bij_ref, wli_ref,
             bli_ref, wlj_ref, blj_ref, wlin_ref, blin_ref, o_log, o_yi,
             o_yj):
        red = jnp.sum(sc_ref[...], axis=0, keepdims=True)
        s_row = red[:, 0:NB]
        c_row = jnp.maximum(red[:, NB : 2 * NB], 1.0)
        mean_row = s_row / c_row

        def mean(b0):
            return mean_row[:, b0 : b0 + NP]

        rv_i = jnp.sum(rii_ref[...] + rji_ref[...], axis=0, keepdims=True)
        rv_j = jnp.sum(rjj_ref[...] + rij_ref[...], axis=0, keepdims=True)
        cb_i = jnp.sum(bii_ref[...]) + jnp.sum(bji_ref[...])
        cb_j = jnp.sum(bjj_ref[...]) + jnp.sum(bij_ref[...])
        xv_i = jnp.dot(rv_i, xiT_ref[...], preferred_element_type=jnp.float32, precision=lax.Precision.HIGHEST)
        xv_j = jnp.dot(rv_j, xjT_ref[...], preferred_element_type=jnp.float32, precision=lax.Precision.HIGHEST)
        xi = jnp.tanh(xv_i + mean(0) + mean(3 * NP) + cb_i)
        xj = jnp.tanh(xv_j + mean(NP) + mean(2 * NP) + cb_j)
        oh_i = (
            lax.broadcasted_iota(jnp.int32, (NP, B), 1) == bci_ref[...]
        ).astype(jnp.float32)
        oh_j = (
            lax.broadcasted_iota(jnp.int32, (NP, B), 1) == bcj_ref[...]
        ).astype(jnp.float32)
        p_i = jnp.tanh(jnp.dot(xi, oh_i, preferred_element_type=jnp.float32, precision=lax.Precision.HIGHEST))
        p_j = jnp.tanh(jnp.dot(xj, oh_j, preferred_element_type=jnp.float32, precision=lax.Precision.HIGHEST))
        o_yi[...] = (
            jnp.dot(wli_ref[...], p_i, preferred_element_type=jnp.float32, precision=lax.Precision.HIGHEST)
            + bli_ref[...]
        )
        o_yj[...] = (
            jnp.dot(wlj_ref[...], p_j, preferred_element_type=jnp.float32, precision=lax.Precision.HIGHEST)
            + blj_ref[...]
        )
        o_log[...] = jax.nn.sigmoid(
            (p_i + p_j) * wlin_ref[...] + blin_ref[...]
        )

    return pl.pallas_call(
        body,
        out_shape=(
            jax.ShapeDtypeStruct((1, B), jnp.float32),
            jax.ShapeDtypeStruct((8, B), jnp.float32),
            jax.ShapeDtypeStruct((8, B), jnp.float32),
        ),
    )(xiT, xjT, sc4, bci, bcj, riiT, rjiT, rjjT, rijT, bii, bji, bjj, bij,
      wliT, bliT, wljT, bljT, wlin, blin)


def kernel(x_i, x_j, edge_index_ii, edge_index_jj, edge_index_ij,
           edge_index_ji, edge_attr_ii, edge_attr_jj, edge_attr_ij,
           edge_attr_ji, batch_i, batch_j, W1, b1, W2, b2, root_ii, bias_ii,
           root_jj, bias_jj, root_ij, bias_ij, root_ji, bias_ji, Wli, bli,
           Wlj, blj, Wlin, blin):
    f32, i32 = jnp.float32, jnp.int32
    pad_e = EP - 4 * E

    ea = jnp.concatenate(
        [edge_attr_ii, edge_attr_jj, edge_attr_ij, edge_attr_ji], axis=0
    )
    ea = jnp.pad(ea, ((0, pad_e), (0, 3)))
    src = jnp.concatenate(
        [
            edge_index_ii[0],
            edge_index_jj[0] + N,
            edge_index_ij[0],
            edge_index_ji[0] + N,
        ]
    ).astype(i32)
    src3 = jnp.pad(src, (0, pad_e)).reshape(NW, NCHUNK, 128)
    dst = jnp.concatenate(
        [
            edge_index_ii[1],
            edge_index_jj[1] + NP,
            edge_index_ij[1] + 2 * NP,
            edge_index_ji[1] + 3 * NP,
        ]
    ).astype(i32)
    dst = jnp.pad(dst, (0, pad_e), constant_values=N)
    table = jnp.concatenate([x_i, x_j], axis=0)

    xs = _sc_gather(table, src3)
    ms2 = _tc_mlp(
        ea,
        xs,
        jnp.pad(W1, ((0, 3), (0, 0))),
        b1.reshape(1, 128),
        W2,
        b2.reshape(1, 256),
    )
    part = _sc_scatter(ms2.reshape(EP), dst)
    sc4 = part.reshape(NW, 2 * NB)

    pad_n = NP - N
    xiT = jnp.pad(x_i, ((0, pad_n), (0, 0))).T
    xjT = jnp.pad(x_j, ((0, pad_n), (0, 0))).T
    bci = jnp.pad(batch_i.astype(i32), (0, pad_n), constant_values=-1)
    bcj = jnp.pad(batch_j.astype(i32), (0, pad_n), constant_values=-1)

    log_row, yiT, yjT = _tc_final(
        xiT,
        xjT,
        sc4,
        bci.reshape(NP, 1),
        bcj.reshape(NP, 1),
        root_ii.T,
        root_ji.T,
        root_jj.T,
        root_ij.T,
        bias_ii.reshape(1, DIM),
        bias_ji.reshape(1, DIM),
        bias_jj.reshape(1, DIM),
        bias_ij.reshape(1, DIM),
        Wli.reshape(8, 1),
        bli.reshape(8, 1),
        Wlj.reshape(8, 1),
        blj.reshape(8, 1),
        Wlin,
        blin.reshape(1, 1),
    )
    return (log_row.reshape(B, 1), yiT.T, yjT.T)


# speedup vs baseline: 4.5786x; 1.8674x over previous
import functools

import jax
import jax.numpy as jnp
from jax import lax
from jax.experimental import pallas as pl
from jax.experimental.pallas import tpu as pltpu
from jax.experimental.pallas import tpu_sc as plsc

N = 10000
E = 160000
DIM = 16
B = 128

NW = 32
EPW = 20480
EP = NW * EPW
NCHUNK = EPW // 128
NP = 10240
NB = 4 * NP

def _mesh():
    return plsc.VectorSubcoreMesh(core_axis_name="c", subcore_axis_name="s")


_sc_params = pltpu.CompilerParams(use_tc_tiling_on_sc=False)
_sc_params_nl = pltpu.CompilerParams(
    use_tc_tiling_on_sc=False, needs_layout_passes=False
)


def _sc_gather(table, idx3):
    @functools.partial(
        pl.kernel,
        mesh=_mesh(),
        out_type=jax.ShapeDtypeStruct((EP, 16), jnp.float32),
        compiler_params=_sc_params,
        scratch_types=[
            pltpu.VMEM((NCHUNK, 128), jnp.int32),
            pltpu.VMEM((2048, 16), jnp.float32),
            pltpu.SemaphoreType.DMA,
        ],
    )
    def gk(table_hbm, idx_hbm, out_hbm, idx_v, rows_v, sem):
        wid = lax.axis_index("s") * 2 + lax.axis_index("c")
        pltpu.sync_copy(idx_hbm.at[wid], idx_v)

        @pl.loop(0, NCHUNK // 16)
        def _(ob):
            copies = []
            for j in range(16):
                copies.append(
                    pltpu.async_copy(
                        table_hbm.at[idx_v.at[ob * 16 + j]],
                        rows_v.at[pl.ds(j * 128, 128)],
                        sem,
                    )
                )
            for cp in copies:
                cp.wait()
            pltpu.sync_copy(
                rows_v, out_hbm.at[pl.ds(wid * EPW + ob * 2048, 2048)]
            )

    return gk(table, idx3)


def _sc_scatter(ms, dst):
    CH = 2560

    @functools.partial(
        pl.kernel,
        mesh=_mesh(),
        out_type=jax.ShapeDtypeStruct((NW, 2, NB), jnp.float32),
        compiler_params=_sc_params_nl,
        scratch_types=[
            pltpu.VMEM((CH,), jnp.int32),
            pltpu.VMEM((CH,), jnp.float32),
            pltpu.VMEM((NB,), jnp.float32),
            pltpu.VMEM((NB,), jnp.float32),
        ],
    )
    def sk(ms_hbm, dst_hbm, out_hbm, idx_v, vals_v, s_acc, c_acc):
        c = lax.axis_index("c")
        s = lax.axis_index("s")
        wid = s * 2 + c
        zero16 = jnp.zeros((16,), jnp.float32)
        ones16 = jnp.ones((16,), jnp.float32)

        @pl.loop(0, NB, step=16)
        def _(i):
            s_acc[pl.ds(i, 16)] = zero16
            c_acc[pl.ds(i, 16)] = zero16

        @pl.loop(0, EPW // CH)
        def _(ob):
            base = wid * EPW + ob * CH
            pltpu.sync_copy(dst_hbm.at[pl.ds(base, CH)], idx_v)
            pltpu.sync_copy(ms_hbm.at[pl.ds(base, CH)], vals_v)

            @pl.loop(0, CH, step=16)
            def _(e):
                iv = idx_v[pl.ds(e, 16)]
                plsc.addupdate_scatter(s_acc, [iv], vals_v[pl.ds(e, 16)])
                plsc.addupdate_scatter(c_acc, [iv], ones16)

        pltpu.sync_copy(s_acc, out_hbm.at[wid, 0])
        pltpu.sync_copy(c_acc, out_hbm.at[wid, 1])

    return sk(ms, dst)


def _tc_mlp(eaT, xs, w1T, b1c, w2T, b2c):
    BE = 10240
    G = EP // BE
    HI = lax.Precision.HIGHEST

    def body(eaT_ref, xs_ref, w1T_ref, b1_ref, w2T_ref, b2_ref, out_ref):
        selT = (
            lax.broadcasted_iota(jnp.int32, (16, 256), 1) // 16
            == lax.broadcasted_iota(jnp.int32, (16, 256), 0)
        ).astype(jnp.float32)
        w2rT = jnp.dot(selT, w2T_ref[...], preferred_element_type=jnp.float32,
                       precision=HI)
        b2rT = jnp.dot(selT, b2_ref[...], preferred_element_type=jnp.float32,
                       precision=HI)
        gT = jnp.maximum(
            jnp.dot(w1T_ref[...], eaT_ref[...],
                    preferred_element_type=jnp.float32, precision=HI)
            + b1_ref[...],
            0.0,
        )
        qT = jnp.dot(w2rT, gT, preferred_element_type=jnp.float32,
                     precision=HI) + b2rT
        xsT = jnp.swapaxes(xs_ref[...], 0, 1)
        out_ref[0] = jnp.sum(qT * xsT, axis=0, keepdims=True)

    return pl.pallas_call(
        body,
        grid=(G,),
        in_specs=[
            pl.BlockSpec((8, BE), lambda i: (0, i)),
            pl.BlockSpec((BE, 16), lambda i: (i, 0)),
            pl.BlockSpec((128, 8), lambda i: (0, 0)),
            pl.BlockSpec((128, 1), lambda i: (0, 0)),
            pl.BlockSpec((256, 128), lambda i: (0, 0)),
            pl.BlockSpec((256, 1), lambda i: (0, 0)),
        ],
        out_specs=pl.BlockSpec((1, 1, BE), lambda i: (i, 0, 0)),
        out_shape=jax.ShapeDtypeStruct((G, 1, BE), jnp.float32),
    )(eaT, xs, w1T, b1c, w2T, b2c)


def _tc_final(xiT, xjT, sc4, bci, bcj, riiT, rjiT, rjjT, rijT, bii, bji, bjj,
              bij, wliT, bliT, wljT, bljT, wlin, blin):
    def body(xiT_ref, xjT_ref, sc_ref, bci_ref, bcj_ref, rii_ref, rji_ref,
             rjj_ref, rij_ref, bii_ref, bji_ref, bjj_ref, bij_ref, wli_ref,
             bli_ref, wlj_ref, blj_ref, wlin_ref, blin_ref, o_log, o_yi,
             o_yj):
        red = jnp.sum(sc_ref[...], axis=0, keepdims=True)
        s_row = red[:, 0:NB]
        c_row = jnp.maximum(red[:, NB : 2 * NB], 1.0)
        mean_row = s_row / c_row

        def mean(b0):
            return mean_row[:, b0 : b0 + NP]

        rv_i = jnp.sum(rii_ref[...] + rji_ref[...], axis=0, keepdims=True)
        rv_j = jnp.sum(rjj_ref[...] + rij_ref[...], axis=0, keepdims=True)
        cb_i = jnp.sum(bii_ref[...]) + jnp.sum(bji_ref[...])
        cb_j = jnp.sum(bjj_ref[...]) + jnp.sum(bij_ref[...])
        xv_i = jnp.dot(rv_i, xiT_ref[...], preferred_element_type=jnp.float32, precision=lax.Precision.HIGHEST)
        xv_j = jnp.dot(rv_j, xjT_ref[...], preferred_element_type=jnp.float32, precision=lax.Precision.HIGHEST)
        xi = jnp.tanh(xv_i + mean(0) + mean(3 * NP) + cb_i)
        xj = jnp.tanh(xv_j + mean(NP) + mean(2 * NP) + cb_j)
        oh_i = (
            lax.broadcasted_iota(jnp.int32, (NP, B), 1) == bci_ref[...]
        ).astype(jnp.float32)
        oh_j = (
            lax.broadcasted_iota(jnp.int32, (NP, B), 1) == bcj_ref[...]
        ).astype(jnp.float32)
        p_i = jnp.tanh(jnp.dot(xi, oh_i, preferred_element_type=jnp.float32, precision=lax.Precision.HIGHEST))
        p_j = jnp.tanh(jnp.dot(xj, oh_j, preferred_element_type=jnp.float32, precision=lax.Precision.HIGHEST))
        o_yi[...] = (
            jnp.dot(wli_ref[...], p_i, preferred_element_type=jnp.float32, precision=lax.Precision.HIGHEST)
            + bli_ref[...]
        )
        o_yj[...] = (
            jnp.dot(wlj_ref[...], p_j, preferred_element_type=jnp.float32, precision=lax.Precision.HIGHEST)
            + blj_ref[...]
        )
        o_log[...] = jax.nn.sigmoid(
            (p_i + p_j) * wlin_ref[...] + blin_ref[...]
        )

    return pl.pallas_call(
        body,
        out_shape=(
            jax.ShapeDtypeStruct((1, B), jnp.float32),
            jax.ShapeDtypeStruct((8, B), jnp.float32),
            jax.ShapeDtypeStruct((8, B), jnp.float32),
        ),
    )(xiT, xjT, sc4, bci, bcj, riiT, rjiT, rjjT, rijT, bii, bji, bjj, bij,
      wliT, bliT, wljT, bljT, wlin, blin)


def kernel(x_i, x_j, edge_index_ii, edge_index_jj, edge_index_ij,
           edge_index_ji, edge_attr_ii, edge_attr_jj, edge_attr_ij,
           edge_attr_ji, batch_i, batch_j, W1, b1, W2, b2, root_ii, bias_ii,
           root_jj, bias_jj, root_ij, bias_ij, root_ji, bias_ji, Wli, bli,
           Wlj, blj, Wlin, blin):
    f32, i32 = jnp.float32, jnp.int32
    pad_e = EP - 4 * E

    ea = jnp.concatenate(
        [edge_attr_ii, edge_attr_jj, edge_attr_ij, edge_attr_ji], axis=0
    )
    ea = jnp.pad(ea, ((0, pad_e), (0, 3)))
    src = jnp.concatenate(
        [
            edge_index_ii[0],
            edge_index_jj[0] + N,
            edge_index_ij[0],
            edge_index_ji[0] + N,
        ]
    ).astype(i32)
    src3 = jnp.pad(src, (0, pad_e)).reshape(NW, NCHUNK, 128)
    dst = jnp.concatenate(
        [
            edge_index_ii[1],
            edge_index_jj[1] + NP,
            edge_index_ij[1] + 2 * NP,
            edge_index_ji[1] + 3 * NP,
        ]
    ).astype(i32)
    dst = jnp.pad(dst, (0, pad_e), constant_values=N)
    table = jnp.concatenate([x_i, x_j], axis=0)

    xs = _sc_gather(table, src3)
    ms2 = _tc_mlp(
        ea.T,
        xs,
        jnp.pad(W1, ((0, 3), (0, 0))).T,
        b1.reshape(128, 1),
        W2.T,
        b2.reshape(256, 1),
    )
    part = _sc_scatter(ms2.reshape(EP), dst)
    sc4 = part.reshape(NW, 2 * NB)

    pad_n = NP - N
    xiT = jnp.pad(x_i, ((0, pad_n), (0, 0))).T
    xjT = jnp.pad(x_j, ((0, pad_n), (0, 0))).T
    bci = jnp.pad(batch_i.astype(i32), (0, pad_n), constant_values=-1)
    bcj = jnp.pad(batch_j.astype(i32), (0, pad_n), constant_values=-1)

    log_row, yiT, yjT = _tc_final(
        xiT,
        xjT,
        sc4,
        bci.reshape(NP, 1),
        bcj.reshape(NP, 1),
        root_ii.T,
        root_ji.T,
        root_jj.T,
        root_ij.T,
        bias_ii.reshape(1, DIM),
        bias_ji.reshape(1, DIM),
        bias_jj.reshape(1, DIM),
        bias_ij.reshape(1, DIM),
        Wli.reshape(8, 1),
        bli.reshape(8, 1),
        Wlj.reshape(8, 1),
        blj.reshape(8, 1),
        Wlin,
        blin.reshape(1, 1),
    )
    return (log_row.reshape(B, 1), yiT.T, yjT.T)


# speedup vs baseline: 4.6708x; 1.0201x over previous
import functools

import jax
import jax.numpy as jnp
from jax import lax
from jax.experimental import pallas as pl
from jax.experimental.pallas import tpu as pltpu
from jax.experimental.pallas import tpu_sc as plsc

N = 10000
E = 160000
DIM = 16
B = 128

NW = 32
EPW = 20480
EP = NW * EPW
NCHUNK = EPW // 128
NP = 10240
NB = 4 * NP

def _mesh():
    return plsc.VectorSubcoreMesh(core_axis_name="c", subcore_axis_name="s")


_sc_params = pltpu.CompilerParams(use_tc_tiling_on_sc=False)
_sc_params_nl = pltpu.CompilerParams(
    use_tc_tiling_on_sc=False, needs_layout_passes=False
)


def _sc_gather(table, idx3):
    nch = idx3.shape[1]
    epw = nch * 128

    @functools.partial(
        pl.kernel,
        mesh=_mesh(),
        out_type=jax.ShapeDtypeStruct((NW * epw, 16), jnp.float32),
        compiler_params=_sc_params,
        scratch_types=[
            pltpu.VMEM((nch, 128), jnp.int32),
            pltpu.VMEM((1024, 16), jnp.float32),
            pltpu.SemaphoreType.DMA,
        ],
    )
    def gk(table_hbm, idx_hbm, out_hbm, idx_v, rows_v, sem):
        wid = lax.axis_index("s") * 2 + lax.axis_index("c")
        pltpu.sync_copy(idx_hbm.at[wid], idx_v)

        @pl.loop(0, nch // 8)
        def _(ob):
            copies = []
            for j in range(8):
                copies.append(
                    pltpu.async_copy(
                        table_hbm.at[idx_v.at[ob * 8 + j]],
                        rows_v.at[pl.ds(j * 128, 128)],
                        sem,
                    )
                )
            for cp in copies:
                cp.wait()
            pltpu.sync_copy(
                rows_v, out_hbm.at[pl.ds(wid * epw + ob * 1024, 1024)]
            )

    return gk(table, idx3)


def _sc_scatter(ms, dst):
    CH = 2560
    epw = ms.shape[0] // NW

    @functools.partial(
        pl.kernel,
        mesh=_mesh(),
        out_type=jax.ShapeDtypeStruct((NW, 2, NB), jnp.float32),
        compiler_params=_sc_params_nl,
        scratch_types=[
            pltpu.VMEM((CH,), jnp.int32),
            pltpu.VMEM((CH,), jnp.float32),
            pltpu.VMEM((NB,), jnp.float32),
            pltpu.VMEM((NB,), jnp.float32),
        ],
    )
    def sk(ms_hbm, dst_hbm, out_hbm, idx_v, vals_v, s_acc, c_acc):
        c = lax.axis_index("c")
        s = lax.axis_index("s")
        wid = s * 2 + c
        zero16 = jnp.zeros((16,), jnp.float32)
        ones16 = jnp.ones((16,), jnp.float32)

        @pl.loop(0, NB, step=16)
        def _(i):
            s_acc[pl.ds(i, 16)] = zero16
            c_acc[pl.ds(i, 16)] = zero16

        @pl.loop(0, epw // CH)
        def _(ob):
            base = wid * epw + ob * CH
            pltpu.sync_copy(dst_hbm.at[pl.ds(base, CH)], idx_v)
            pltpu.sync_copy(ms_hbm.at[pl.ds(base, CH)], vals_v)

            @pl.loop(0, CH, step=16)
            def _(e):
                iv = idx_v[pl.ds(e, 16)]
                plsc.addupdate_scatter(s_acc, [iv], vals_v[pl.ds(e, 16)])
                plsc.addupdate_scatter(c_acc, [iv], ones16)

        pltpu.sync_copy(s_acc, out_hbm.at[wid, 0])
        pltpu.sync_copy(c_acc, out_hbm.at[wid, 1])

    return sk(ms, dst)


def _tc_mlp(eaT, xs, w1T, b1c, w2T, b2c):
    BE = 10240
    G = eaT.shape[1] // BE
    HI = lax.Precision.HIGHEST

    def body(eaT_ref, xs_ref, w1T_ref, b1_ref, w2T_ref, b2_ref, out_ref):
        selT = (
            lax.broadcasted_iota(jnp.int32, (16, 256), 1) // 16
            == lax.broadcasted_iota(jnp.int32, (16, 256), 0)
        ).astype(jnp.float32)
        w2rT = jnp.dot(selT, w2T_ref[...], preferred_element_type=jnp.float32,
                       precision=HI)
        b2rT = jnp.dot(selT, b2_ref[...], preferred_element_type=jnp.float32,
                       precision=HI)
        gT = jnp.maximum(
            jnp.dot(w1T_ref[...], eaT_ref[...],
                    preferred_element_type=jnp.float32, precision=HI)
            + b1_ref[...],
            0.0,
        )
        qT = jnp.dot(w2rT, gT, preferred_element_type=jnp.float32,
                     precision=HI) + b2rT
        xsT = jnp.swapaxes(xs_ref[...], 0, 1)
        out_ref[0] = jnp.sum(qT * xsT, axis=0, keepdims=True)

    return pl.pallas_call(
        body,
        grid=(G,),
        in_specs=[
            pl.BlockSpec((8, BE), lambda i: (0, i)),
            pl.BlockSpec((BE, 16), lambda i: (i, 0)),
            pl.BlockSpec((128, 8), lambda i: (0, 0)),
            pl.BlockSpec((128, 1), lambda i: (0, 0)),
            pl.BlockSpec((256, 128), lambda i: (0, 0)),
            pl.BlockSpec((256, 1), lambda i: (0, 0)),
        ],
        out_specs=pl.BlockSpec((1, 1, BE), lambda i: (i, 0, 0)),
        out_shape=jax.ShapeDtypeStruct((G, 1, BE), jnp.float32),
    )(eaT, xs, w1T, b1c, w2T, b2c)


def _tc_final(xiT, xjT, sc4, bci, bcj, riiT, rjiT, rjjT, rijT, bii, bji, bjj,
              bij, wliT, bliT, wljT, bljT, wlin, blin):
    def body(xiT_ref, xjT_ref, sc_ref, bci_ref, bcj_ref, rii_ref, rji_ref,
             rjj_ref, rij_ref, bii_ref, bji_ref, bjj_ref, bij_ref, wli_ref,
             bli_ref, wlj_ref, blj_ref, wlin_ref, blin_ref, o_log, o_yi,
             o_yj):
        red = jnp.sum(sc_ref[...], axis=0, keepdims=True)
        s_row = red[:, 0:NB]
        c_row = jnp.maximum(red[:, NB : 2 * NB], 1.0)
        mean_row = s_row / c_row

        def mean(b0):
            return mean_row[:, b0 : b0 + NP]

        rv_i = jnp.sum(rii_ref[...] + rji_ref[...], axis=0, keepdims=True)
        rv_j = jnp.sum(rjj_ref[...] + rij_ref[...], axis=0, keepdims=True)
        cb_i = jnp.sum(bii_ref[...]) + jnp.sum(bji_ref[...])
        cb_j = jnp.sum(bjj_ref[...]) + jnp.sum(bij_ref[...])
        xv_i = jnp.dot(rv_i, xiT_ref[...], preferred_element_type=jnp.float32, precision=lax.Precision.HIGHEST)
        xv_j = jnp.dot(rv_j, xjT_ref[...], preferred_element_type=jnp.float32, precision=lax.Precision.HIGHEST)
        xi = jnp.tanh(xv_i + mean(0) + mean(3 * NP) + cb_i)
        xj = jnp.tanh(xv_j + mean(NP) + mean(2 * NP) + cb_j)
        oh_i = (
            lax.broadcasted_iota(jnp.int32, (NP, B), 1) == bci_ref[...]
        ).astype(jnp.float32)
        oh_j = (
            lax.broadcasted_iota(jnp.int32, (NP, B), 1) == bcj_ref[...]
        ).astype(jnp.float32)
        p_i = jnp.tanh(jnp.dot(xi, oh_i, preferred_element_type=jnp.float32, precision=lax.Precision.HIGHEST))
        p_j = jnp.tanh(jnp.dot(xj, oh_j, preferred_element_type=jnp.float32, precision=lax.Precision.HIGHEST))
        o_yi[...] = (
            jnp.dot(wli_ref[...], p_i, preferred_element_type=jnp.float32, precision=lax.Precision.HIGHEST)
            + bli_ref[...]
        )
        o_yj[...] = (
            jnp.dot(wlj_ref[...], p_j, preferred_element_type=jnp.float32, precision=lax.Precision.HIGHEST)
            + blj_ref[...]
        )
        o_log[...] = jax.nn.sigmoid(
            (p_i + p_j) * wlin_ref[...] + blin_ref[...]
        )

    return pl.pallas_call(
        body,
        out_shape=(
            jax.ShapeDtypeStruct((1, B), jnp.float32),
            jax.ShapeDtypeStruct((8, B), jnp.float32),
            jax.ShapeDtypeStruct((8, B), jnp.float32),
        ),
    )(xiT, xjT, sc4, bci, bcj, riiT, rjiT, rjjT, rijT, bii, bji, bjj, bij,
      wliT, bliT, wljT, bljT, wlin, blin)


def kernel(x_i, x_j, edge_index_ii, edge_index_jj, edge_index_ij,
           edge_index_ji, edge_attr_ii, edge_attr_jj, edge_attr_ij,
           edge_attr_ji, batch_i, batch_j, W1, b1, W2, b2, root_ii, bias_ii,
           root_jj, bias_jj, root_ij, bias_ij, root_ji, bias_ji, Wli, bli,
           Wlj, blj, Wlin, blin):
    f32, i32 = jnp.float32, jnp.int32
    pad_e = EP - 4 * E

    ea = jnp.concatenate(
        [edge_attr_ii, edge_attr_jj, edge_attr_ij, edge_attr_ji], axis=0
    )
    ea = jnp.pad(ea, ((0, pad_e), (0, 3)))
    src = jnp.concatenate(
        [
            edge_index_ii[0],
            edge_index_jj[0] + N,
            edge_index_ij[0],
            edge_index_ji[0] + N,
        ]
    ).astype(i32)
    src = jnp.pad(src, (0, pad_e))
    dst = jnp.concatenate(
        [
            edge_index_ii[1],
            edge_index_jj[1] + NP,
            edge_index_ij[1] + 2 * NP,
            edge_index_ji[1] + 3 * NP,
        ]
    ).astype(i32)
    dst = jnp.pad(dst, (0, pad_e), constant_values=N)
    table = jnp.concatenate([x_i, x_j], axis=0)

    eaT = ea.T
    w1T = jnp.pad(W1, ((0, 3), (0, 0))).T
    b1c = b1.reshape(128, 1)
    w2T = W2.T
    b2c = b2.reshape(256, 1)

    S = 2
    EP2 = EP // S
    parts = []
    for h in range(S):
        sl = slice(h * EP2, (h + 1) * EP2)
        src3 = src[sl].reshape(NW, EP2 // NW // 128, 128)
        xs = _sc_gather(table, src3)
        msh = _tc_mlp(eaT[:, sl], xs, w1T, b1c, w2T, b2c)
        parts.append(_sc_scatter(msh.reshape(EP2), dst[sl]))
    sc4 = jnp.concatenate(parts).reshape(S * NW, 2 * NB)

    pad_n = NP - N
    xiT = jnp.pad(x_i, ((0, pad_n), (0, 0))).T
    xjT = jnp.pad(x_j, ((0, pad_n), (0, 0))).T
    bci = jnp.pad(batch_i.astype(i32), (0, pad_n), constant_values=-1)
    bcj = jnp.pad(batch_j.astype(i32), (0, pad_n), constant_values=-1)

    log_row, yiT, yjT = _tc_final(
        xiT,
        xjT,
        sc4,
        bci.reshape(NP, 1),
        bcj.reshape(NP, 1),
        root_ii.T,
        root_ji.T,
        root_jj.T,
        root_ij.T,
        bias_ii.reshape(1, DIM),
        bias_ji.reshape(1, DIM),
        bias_jj.reshape(1, DIM),
        bias_ij.reshape(1, DIM),
        Wli.reshape(8, 1),
        bli.reshape(8, 1),
        Wlj.reshape(8, 1),
        blj.reshape(8, 1),
        Wlin,
        blin.reshape(1, 1),
    )
    return (log_row.reshape(B, 1), yiT.T, yjT.T)
